# Initial kernel scaffold; baseline (speedup 1.0000x reference)
import jax
import jax.numpy as jnp
from jax.experimental import pallas as pl


def kernel(coords):
    raise NotImplementedError("write your pallas kernel here")


# speedup vs baseline: 2.2313x; 2.2313x over previous
import jax
import jax.numpy as jnp
from jax.experimental import pallas as pl
from jax.experimental.pallas import tpu as pltpu

_N = 10000
_K = 32
_KK = _K + 1
_RT = 128
_NP = 10112
_GRID = _NP // _RT
_OW = 128


def _knn_kernel(cq_ref, cc_ref, cct_ref, idx_ref, w_ref, idx0_ref, w0_ref):
    pid = pl.program_id(0)

    @pl.when(pid == 0)
    def _depot_row():
        xs = cct_ref[0:1, :]
        ys = cct_ref[1:2, :]
        dx = xs - cct_ref[0, 0]
        dy = ys - cct_ref[1, 0]
        s = dx * dx + dy * dy
        w = jnp.where(s > 0.0, jnp.sqrt(jnp.where(s > 0.0, s, 1.0)), 0.0)
        col = jax.lax.broadcasted_iota(jnp.int32, (1, _NP), 1)
        w = jnp.where(col < _N, w, jnp.inf)
        lane = jax.lax.broadcasted_iota(jnp.int32, (1, _OW), 1)

        def body(i, carry):
            w, acc_i, acc_w = carry
            m = jnp.min(w)
            a = jnp.min(jnp.where(w == m, col, jnp.int32(2**30)))
            acc_i = jnp.where(lane == i, a, acc_i)
            acc_w = jnp.where(lane == i, m, acc_w)
            w = jnp.where(col == a, jnp.inf, w)
            return w, acc_i, acc_w

        acc_i0 = jnp.zeros((1, _OW), jnp.int32)
        acc_w0 = jnp.zeros((1, _OW), jnp.float32)
        _, acc_i, acc_w = jax.lax.fori_loop(0, _KK, body, (w, acc_i0, acc_w0))
        idx0_ref[0:1, :] = acc_i
        w0_ref[0:1, :] = acc_w

    xq = cq_ref[:, :]
    xc = cc_ref[:, :]
    xq0 = xq[:, 0]
    xq1 = xq[:, 1]
    xc0 = xc[:, 0]
    xc1 = xc[:, 1]
    sq_q = xq0 * xq0 + xq1 * xq1
    sq_c = xc0 * xc0 + xc1 * xc1
    bq0 = xq0.astype(jnp.bfloat16).astype(jnp.float32)
    bq1 = xq1.astype(jnp.bfloat16).astype(jnp.float32)
    bc0 = xc0.astype(jnp.bfloat16).astype(jnp.float32)
    bc1 = xc1.astype(jnp.bfloat16).astype(jnp.float32)
    cross_b = bq0[:, None] * bc0[None, :] + bq1[:, None] * bc1[None, :]
    key = (sq_q[:, None] + sq_c[None, :]) - 2.0 * cross_b
    cross = xq0[:, None] * xc0[None, :] + xq1[:, None] * xc1[None, :]
    d2 = (sq_q[:, None] + sq_c[None, :]) - 2.0 * cross

    col = jax.lax.broadcasted_iota(jnp.int32, (_RT, _NP), 1)
    valid = (col >= 1) & (col < _N)
    key = jnp.where(valid, key, jnp.inf)

    rowt = pid * _RT + jax.lax.broadcasted_iota(jnp.int32, (_RT, 1), 0) + 1
    lane = jax.lax.broadcasted_iota(jnp.int32, (_RT, _OW), 1)

    def body(i, carry):
        key, acc_i, acc_w = carry
        m = jnp.min(key, axis=1, keepdims=True)
        a = jnp.min(jnp.where(key == m, col, jnp.int32(2**30)),
                    axis=1, keepdims=True)
        val = jnp.sum(jnp.where(col == a, d2, 0.0), axis=1, keepdims=True)
        wv = jnp.sqrt(jnp.maximum(val, 0.0))
        wv = jnp.where(a == rowt, 0.0, wv)
        acc_i = jnp.where(lane == i, a, acc_i)
        acc_w = jnp.where(lane == i, wv, acc_w)
        key = jnp.where(col == a, jnp.inf, key)
        return key, acc_i, acc_w

    acc_i0 = jnp.zeros((_RT, _OW), jnp.int32)
    acc_w0 = jnp.zeros((_RT, _OW), jnp.float32)
    _, acc_i, acc_w = jax.lax.fori_loop(0, _K, body, (key, acc_i0, acc_w0))

    depx = cc_ref[0, 0]
    depy = cc_ref[0, 1]
    ddx = xq0 - depx
    ddy = xq1 - depy
    s = (ddx * ddx + ddy * ddy)[:, None]
    wd = jnp.where(s > 0.0, jnp.sqrt(jnp.where(s > 0.0, s, 1.0)), 0.0)
    acc_i = jnp.where(lane == _K, 0, acc_i)
    acc_w = jnp.where(lane == _K, wd, acc_w)
    idx_ref[:, :] = acc_i
    w_ref[:, :] = acc_w


def kernel(coords):
    n = _N
    cq = jnp.zeros((_NP, 2), jnp.float32).at[: n - 1].set(coords[1:])
    cc = jnp.zeros((_NP, 2), jnp.float32).at[:n].set(coords)
    cct = cc.T

    idx, w, idx0, w0 = pl.pallas_call(
        _knn_kernel,
        grid=(_GRID,),
        in_specs=[
            pl.BlockSpec((_RT, 2), lambda i: (i, 0)),
            pl.BlockSpec((_NP, 2), lambda i: (0, 0)),
            pl.BlockSpec((2, _NP), lambda i: (0, 0)),
        ],
        out_specs=[
            pl.BlockSpec((_RT, _OW), lambda i: (i, 0)),
            pl.BlockSpec((_RT, _OW), lambda i: (i, 0)),
            pl.BlockSpec((8, _OW), lambda i: (0, 0)),
            pl.BlockSpec((8, _OW), lambda i: (0, 0)),
        ],
        out_shape=[
            jax.ShapeDtypeStruct((_NP, _OW), jnp.int32),
            jax.ShapeDtypeStruct((_NP, _OW), jnp.float32),
            jax.ShapeDtypeStruct((8, _OW), jnp.int32),
            jax.ShapeDtypeStruct((8, _OW), jnp.float32),
        ],
    )(cq, cc, cct)

    src = jnp.concatenate([idx0[0:1, :_KK], idx[: n - 1, :_KK]], axis=0)
    dst = jnp.broadcast_to(jnp.arange(n, dtype=jnp.int32)[:, None], (n, _KK))
    weights = jnp.concatenate([w0[0:1, :_KK], w[: n - 1, :_KK]], axis=0)
    edge_idx = jnp.stack([src, dst], axis=0)
    return edge_idx, weights, _KK


# speedup vs baseline: 2.3211x; 1.0402x over previous
import jax
import jax.numpy as jnp
from jax.experimental import pallas as pl
from jax.experimental.pallas import tpu as pltpu

_N = 10000
_K = 32
_KK = _K + 1
_RT = 128
_NP = 10112
_GRID = _NP // _RT
_OW = 128


def _knn_kernel(cq_ref, cc_ref, cct_ref, idx_ref, w_ref, idx0_ref, w0_ref):
    pid = pl.program_id(0)

    @pl.when(pid == 0)
    def _depot_row():
        xs = cct_ref[0:1, :]
        ys = cct_ref[1:2, :]
        dx = xs - cct_ref[0, 0]
        dy = ys - cct_ref[1, 0]
        s = dx * dx + dy * dy
        w = jnp.where(s > 0.0, jnp.sqrt(jnp.where(s > 0.0, s, 1.0)), 0.0)
        col = jax.lax.broadcasted_iota(jnp.int32, (1, _NP), 1)
        w = jnp.where(col < _N, w, jnp.inf)
        lane = jax.lax.broadcasted_iota(jnp.int32, (1, _OW), 1)

        def body(i, carry):
            w, acc_i, acc_w = carry
            m = jnp.min(w)
            a = jnp.min(jnp.where(w == m, col, jnp.int32(2**30)))
            acc_i = jnp.where(lane == i, a, acc_i)
            acc_w = jnp.where(lane == i, m, acc_w)
            w = jnp.where(col == a, jnp.inf, w)
            return w, acc_i, acc_w

        acc_i0 = jnp.zeros((1, _OW), jnp.int32)
        acc_w0 = jnp.zeros((1, _OW), jnp.float32)
        _, acc_i, acc_w = jax.lax.fori_loop(0, _KK, body, (w, acc_i0, acc_w0))
        idx0_ref[0:1, :] = acc_i
        w0_ref[0:1, :] = acc_w

    xq = cq_ref[:, :]
    xc = cc_ref[:, :]
    xq0 = xq[:, 0]
    xq1 = xq[:, 1]
    xc0 = xc[:, 0]
    xc1 = xc[:, 1]
    sq_q = xq0 * xq0 + xq1 * xq1
    sq_c = xc0 * xc0 + xc1 * xc1
    bq0 = xq0.astype(jnp.bfloat16).astype(jnp.float32)
    bq1 = xq1.astype(jnp.bfloat16).astype(jnp.float32)
    bc0 = xc0.astype(jnp.bfloat16).astype(jnp.float32)
    bc1 = xc1.astype(jnp.bfloat16).astype(jnp.float32)
    cross_b = bq0[:, None] * bc0[None, :] + bq1[:, None] * bc1[None, :]
    key = (sq_q[:, None] + sq_c[None, :]) - 2.0 * cross_b
    cross = xq0[:, None] * xc0[None, :] + xq1[:, None] * xc1[None, :]
    d2 = (sq_q[:, None] + sq_c[None, :]) - 2.0 * cross

    col = jax.lax.broadcasted_iota(jnp.int32, (_RT, _NP), 1)
    valid = (col >= 1) & (col < _N)
    key = jnp.where(valid, key, jnp.inf)

    rowt = pid * _RT + jax.lax.broadcasted_iota(jnp.int32, (_RT, 1), 0) + 1
    lane = jax.lax.broadcasted_iota(jnp.int32, (_RT, _OW), 1)

    def body(i, carry):
        key, acc_i, acc_w = carry
        a = jnp.argmin(key, axis=1).astype(jnp.int32)[:, None]
        val = jnp.sum(jnp.where(col == a, d2, 0.0), axis=1, keepdims=True)
        wv = jnp.sqrt(jnp.maximum(val, 0.0))
        wv = jnp.where(a == rowt, 0.0, wv)
        acc_i = jnp.where(lane == i, a, acc_i)
        acc_w = jnp.where(lane == i, wv, acc_w)
        key = jnp.where(col == a, jnp.inf, key)
        return key, acc_i, acc_w

    acc_i0 = jnp.zeros((_RT, _OW), jnp.int32)
    acc_w0 = jnp.zeros((_RT, _OW), jnp.float32)
    _, acc_i, acc_w = jax.lax.fori_loop(0, _K, body, (key, acc_i0, acc_w0))

    depx = cc_ref[0, 0]
    depy = cc_ref[0, 1]
    ddx = xq0 - depx
    ddy = xq1 - depy
    s = (ddx * ddx + ddy * ddy)[:, None]
    wd = jnp.where(s > 0.0, jnp.sqrt(jnp.where(s > 0.0, s, 1.0)), 0.0)
    acc_i = jnp.where(lane == _K, 0, acc_i)
    acc_w = jnp.where(lane == _K, wd, acc_w)
    idx_ref[:, :] = acc_i
    w_ref[:, :] = acc_w


def kernel(coords):
    n = _N
    cq = jnp.zeros((_NP, 2), jnp.float32).at[: n - 1].set(coords[1:])
    cc = jnp.zeros((_NP, 2), jnp.float32).at[:n].set(coords)
    cct = cc.T

    idx, w, idx0, w0 = pl.pallas_call(
        _knn_kernel,
        grid=(_GRID,),
        in_specs=[
            pl.BlockSpec((_RT, 2), lambda i: (i, 0)),
            pl.BlockSpec((_NP, 2), lambda i: (0, 0)),
            pl.BlockSpec((2, _NP), lambda i: (0, 0)),
        ],
        out_specs=[
            pl.BlockSpec((_RT, _OW), lambda i: (i, 0)),
            pl.BlockSpec((_RT, _OW), lambda i: (i, 0)),
            pl.BlockSpec((8, _OW), lambda i: (0, 0)),
            pl.BlockSpec((8, _OW), lambda i: (0, 0)),
        ],
        out_shape=[
            jax.ShapeDtypeStruct((_NP, _OW), jnp.int32),
            jax.ShapeDtypeStruct((_NP, _OW), jnp.float32),
            jax.ShapeDtypeStruct((8, _OW), jnp.int32),
            jax.ShapeDtypeStruct((8, _OW), jnp.float32),
        ],
    )(cq, cc, cct)

    src = jnp.concatenate([idx0[0:1, :_KK], idx[: n - 1, :_KK]], axis=0)
    dst = jnp.broadcast_to(jnp.arange(n, dtype=jnp.int32)[:, None], (n, _KK))
    weights = jnp.concatenate([w0[0:1, :_KK], w[: n - 1, :_KK]], axis=0)
    edge_idx = jnp.stack([src, dst], axis=0)
    return edge_idx, weights, _KK


# speedup vs baseline: 2.8833x; 1.2422x over previous
import jax
import jax.numpy as jnp
from jax.experimental import pallas as pl
from jax.experimental.pallas import tpu as pltpu
from jax.experimental.pallas import tpu_sc as plsc

_N = 10000
_K = 32
_KK = _K + 1
_RT = 128
_NP = 10112
_GRID = _NP // _RT
_OW = 128

_NW = 32
_B = 330752
_BW = _B // _NW
_TBL = 10240


def _knn_kernel(cq_ref, cc_ref, cct_ref, idx_ref, idx0_ref):
    pid = pl.program_id(0)

    @pl.when(pid == 0)
    def _depot_row():
        xs = cct_ref[0:1, :]
        ys = cct_ref[1:2, :]
        dx = xs - cct_ref[0, 0]
        dy = ys - cct_ref[1, 0]
        s = dx * dx + dy * dy
        w = jnp.where(s > 0.0, jnp.sqrt(jnp.where(s > 0.0, s, 1.0)), 0.0)
        col = jax.lax.broadcasted_iota(jnp.int32, (1, _NP), 1)
        w = jnp.where(col < _N, w, jnp.inf)
        lane = jax.lax.broadcasted_iota(jnp.int32, (1, _OW), 1)

        def body(i, carry):
            w, acc = carry
            m = jnp.min(w)
            a = jnp.min(jnp.where(w == m, col, jnp.int32(2**30)))
            acc = jnp.where(lane == i, a, acc)
            w = jnp.where(col == a, jnp.inf, w)
            return w, acc

        _, acc = jax.lax.fori_loop(
            0, _KK, body, (w, jnp.zeros((1, _OW), jnp.int32)))
        idx0_ref[0:1, :] = acc

    xq = cq_ref[:, :]
    xc = cc_ref[:, :]
    xq0 = xq[:, 0]
    xq1 = xq[:, 1]
    xc0 = xc[:, 0]
    xc1 = xc[:, 1]
    sq_q = xq0 * xq0 + xq1 * xq1
    sq_c = xc0 * xc0 + xc1 * xc1
    bq0 = xq0.astype(jnp.bfloat16).astype(jnp.float32)
    bq1 = xq1.astype(jnp.bfloat16).astype(jnp.float32)
    bc0 = xc0.astype(jnp.bfloat16).astype(jnp.float32)
    bc1 = xc1.astype(jnp.bfloat16).astype(jnp.float32)
    cross_b = bq0[:, None] * bc0[None, :] + bq1[:, None] * bc1[None, :]
    key = (sq_q[:, None] + sq_c[None, :]) - 2.0 * cross_b

    col = jax.lax.broadcasted_iota(jnp.int32, (_RT, _NP), 1)
    valid = (col >= 1) & (col < _N)
    key = jnp.where(valid, key, jnp.inf)
    lane = jax.lax.broadcasted_iota(jnp.int32, (_RT, _OW), 1)

    def body(i, carry):
        key, acc = carry
        m = jnp.min(key, axis=1, keepdims=True)
        a = jnp.min(jnp.where(key == m, col, jnp.int32(2**30)),
                    axis=1, keepdims=True)
        acc = jnp.where(lane == i, a, acc)
        key = jnp.where(col == a, jnp.inf, key)
        return key, acc

    _, acc = jax.lax.fori_loop(
        0, _K, body, (key, jnp.zeros((_RT, _OW), jnp.int32)))
    acc = jnp.where(lane == _K, 0, acc)
    idx_ref[:, :] = acc


def _gather_kernel(cx_hbm, cy_hbm, src_hbm, xs_hbm, ys_hbm,
                   cx_v, cy_v, idx_v, xs_v, ys_v):
    wid = jax.lax.axis_index("s") * 2 + jax.lax.axis_index("c")
    base = wid * _BW
    pltpu.sync_copy(cx_hbm, cx_v)
    pltpu.sync_copy(cy_hbm, cy_v)
    pltpu.sync_copy(src_hbm.at[pl.ds(base, _BW)], idx_v)

    def body(j, carry):
        sl = pl.ds(j * 16, 16)
        ids = idx_v[sl]
        xs_v[sl] = plsc.load_gather(cx_v, [ids])
        ys_v[sl] = plsc.load_gather(cy_v, [ids])
        return carry

    jax.lax.fori_loop(0, _BW // 16, body, 0)
    pltpu.sync_copy(xs_v, xs_hbm.at[pl.ds(base, _BW)])
    pltpu.sync_copy(ys_v, ys_hbm.at[pl.ds(base, _BW)])


def _norm_kernel(xs_ref, ys_ref, cq_ref, w_ref):
    xq = cq_ref[:, 0:1]
    yq = cq_ref[:, 1:2]
    dx = xs_ref[:, :] - xq
    dy = ys_ref[:, :] - yq
    s = dx * dx + dy * dy
    w_ref[:, :] = jnp.where(s > 0.0, jnp.sqrt(jnp.where(s > 0.0, s, 1.0)),
                            0.0)


def kernel(coords):
    n = _N
    cq = jnp.zeros((_NP, 2), jnp.float32).at[: n - 1].set(coords[1:])
    cc = jnp.zeros((_NP, 2), jnp.float32).at[:n].set(coords)
    cct = cc.T

    idx, idx0 = pl.pallas_call(
        _knn_kernel,
        grid=(_GRID,),
        in_specs=[
            pl.BlockSpec((_RT, 2), lambda i: (i, 0)),
            pl.BlockSpec((_NP, 2), lambda i: (0, 0)),
            pl.BlockSpec((2, _NP), lambda i: (0, 0)),
        ],
        out_specs=[
            pl.BlockSpec((_RT, _OW), lambda i: (i, 0)),
            pl.BlockSpec((8, _OW), lambda i: (0, 0)),
        ],
        out_shape=[
            jax.ShapeDtypeStruct((_NP, _OW), jnp.int32),
            jax.ShapeDtypeStruct((8, _OW), jnp.int32),
        ],
    )(cq, cc, cct)

    src = jnp.concatenate([idx0[0:1, :_KK], idx[: n - 1, :_KK]], axis=0)

    cx = jnp.zeros((_TBL,), jnp.float32).at[:n].set(coords[:, 0])
    cy = jnp.zeros((_TBL,), jnp.float32).at[:n].set(coords[:, 1])
    src_flat = jnp.zeros((_B,), jnp.int32).at[: n * _KK].set(
        src.reshape(-1))

    mesh = plsc.VectorSubcoreMesh(core_axis_name="c", subcore_axis_name="s")
    xs, ys = pl.kernel(
        _gather_kernel,
        out_type=[
            jax.ShapeDtypeStruct((_B,), jnp.float32),
            jax.ShapeDtypeStruct((_B,), jnp.float32),
        ],
        mesh=mesh,
        scratch_types=[
            pltpu.VMEM((_TBL,), jnp.float32),
            pltpu.VMEM((_TBL,), jnp.float32),
            pltpu.VMEM((_BW,), jnp.int32),
            pltpu.VMEM((_BW,), jnp.float32),
            pltpu.VMEM((_BW,), jnp.float32),
        ],
        compiler_params=pltpu.CompilerParams(needs_layout_passes=False),
    )(cx, cy, src_flat)

    xs2 = jnp.zeros((_NP, _KK), jnp.float32).at[:n].set(
        xs[: n * _KK].reshape(n, _KK))
    ys2 = jnp.zeros((_NP, _KK), jnp.float32).at[:n].set(
        ys[: n * _KK].reshape(n, _KK))

    weights = pl.pallas_call(
        _norm_kernel,
        grid=(_GRID,),
        in_specs=[
            pl.BlockSpec((_RT, _KK), lambda i: (i, 0)),
            pl.BlockSpec((_RT, _KK), lambda i: (i, 0)),
            pl.BlockSpec((_RT, 2), lambda i: (i, 0)),
        ],
        out_specs=pl.BlockSpec((_RT, _KK), lambda i: (i, 0)),
        out_shape=jax.ShapeDtypeStruct((_NP, _KK), jnp.float32),
    )(xs2, ys2, cc)[:n]

    dst = jnp.broadcast_to(jnp.arange(n, dtype=jnp.int32)[:, None], (n, _KK))
    edge_idx = jnp.stack([src, dst], axis=0)
    return edge_idx, weights, _KK


# speedup vs baseline: 6.3852x; 2.2145x over previous
import jax
import jax.numpy as jnp
from jax.experimental import pallas as pl
from jax.experimental.pallas import tpu as pltpu
from jax.experimental.pallas import tpu_sc as plsc

_N = 10000
_K = 32
_KK = _K + 1
_RT = 128
_NP = 10112
_GRID = _NP // _RT
_OW = 128
_NB = _NP // 128
_P = 6

_NW = 32
_B = 330752
_BW = _B // _NW
_TBL = 10240


def _knn_kernel(cq_ref, cct_ref, cx3_ref, cy3_ref, idx_ref, idx0_ref):
    pid = pl.program_id(0)

    @pl.when(pid == 0)
    def _depot_row():
        xs = cct_ref[0:1, :]
        ys = cct_ref[1:2, :]
        dx = xs - cct_ref[0, 0]
        dy = ys - cct_ref[1, 0]
        s = dx * dx + dy * dy
        w = jnp.where(s > 0.0, jnp.sqrt(jnp.where(s > 0.0, s, 1.0)), 0.0)
        col = jax.lax.broadcasted_iota(jnp.int32, (1, _NP), 1)
        w = jnp.where(col < _N, w, jnp.inf)
        lane = jax.lax.broadcasted_iota(jnp.int32, (1, _OW), 1)

        def body(i, carry):
            w, acc = carry
            m = jnp.min(w)
            a = jnp.min(jnp.where(w == m, col, jnp.int32(2**30)))
            acc = jnp.where(lane == i, a, acc)
            w = jnp.where(col == a, jnp.inf, w)
            return w, acc

        _, acc = jax.lax.fori_loop(
            0, _KK, body, (w, jnp.zeros((1, _OW), jnp.int32)))
        idx0_ref[0:1, :] = acc

    xq = cq_ref[:, :]
    xq0 = xq[:, 0]
    xq1 = xq[:, 1]
    xc0 = cx3_ref[:, :]
    xc1 = cy3_ref[:, :]
    sq_q = xq0 * xq0 + xq1 * xq1
    sq_c = xc0 * xc0 + xc1 * xc1
    bq0 = xq0.astype(jnp.bfloat16).astype(jnp.float32)
    bq1 = xq1.astype(jnp.bfloat16).astype(jnp.float32)
    bc0 = xc0.astype(jnp.bfloat16).astype(jnp.float32)
    bc1 = xc1.astype(jnp.bfloat16).astype(jnp.float32)

    col3 = (jax.lax.broadcasted_iota(jnp.int32, (_RT, _NB, 128), 1) * 128
            + jax.lax.broadcasted_iota(jnp.int32, (_RT, _NB, 128), 2))
    valid = (col3 >= 1) & (col3 < _N)

    def build_key():
        cross = (bq0[:, None, None] * bc0[None, :, :]
                 + bq1[:, None, None] * bc1[None, :, :])
        key = (sq_q[:, None, None] + sq_c[None, :, :]) - 2.0 * cross
        return jnp.where(valid, key, jnp.inf)

    key3 = build_key()
    lane3 = jax.lax.broadcasted_iota(jnp.int32, (_RT, _NB, 128), 2)
    big = jnp.int32(2**30)

    cks, ccs = [], []
    for _p in range(_P):
        bm = jnp.min(key3, axis=2)
        abm = jnp.min(jnp.where(key3 == bm[:, :, None], col3, big), axis=2)
        cks.append(bm)
        ccs.append(abm)
        key3 = jnp.where(col3 == abm[:, :, None], jnp.inf, key3)
    bm_final = jnp.min(key3, axis=2)
    ck = jnp.concatenate(cks, axis=1)
    cc = jnp.concatenate(ccs, axis=1)

    lane = jax.lax.broadcasted_iota(jnp.int32, (_RT, _OW), 1)

    def body(i, carry):
        ck, acc, _ = carry
        m = jnp.min(ck, axis=1, keepdims=True)
        a = jnp.min(jnp.where(ck == m, cc, big), axis=1, keepdims=True)
        acc = jnp.where(lane == i, a, acc)
        ck = jnp.where(cc == a, jnp.inf, ck)
        return ck, acc, m

    _, acc, m32 = jax.lax.fori_loop(
        0, _K, body,
        (ck, jnp.zeros((_RT, _OW), jnp.int32), jnp.zeros((_RT, 1),
                                                         jnp.float32)))
    acc = jnp.where(lane == _K, 0, acc)
    idx_ref[:, :] = acc

    fail = jnp.any(bm_final <= m32)

    @pl.when(fail)
    def _fallback():
        key3 = build_key()

        def fbody(i, carry):
            key3, acc = carry
            m = jnp.min(key3, axis=(1, 2))[:, None, None]
            a = jnp.min(jnp.where(key3 == m, col3, big),
                        axis=(1, 2))[:, None, None]
            acc = jnp.where(lane == i, a[:, :, 0], acc)
            key3 = jnp.where(col3 == a, jnp.inf, key3)
            return key3, acc

        _, acc_fb = jax.lax.fori_loop(
            0, _K, fbody, (key3, jnp.zeros((_RT, _OW), jnp.int32)))
        acc_fb = jnp.where(lane == _K, 0, acc_fb)
        idx_ref[:, :] = acc_fb


def _gather_kernel(cx_hbm, cy_hbm, src_hbm, xs_hbm, ys_hbm,
                   cx_v, cy_v, idx_v, xs_v, ys_v):
    wid = jax.lax.axis_index("s") * 2 + jax.lax.axis_index("c")
    base = wid * _BW
    pltpu.sync_copy(cx_hbm, cx_v)
    pltpu.sync_copy(cy_hbm, cy_v)
    pltpu.sync_copy(src_hbm.at[pl.ds(base, _BW)], idx_v)

    def body(j, carry):
        sl = pl.ds(j * 16, 16)
        ids = idx_v[sl]
        xs_v[sl] = plsc.load_gather(cx_v, [ids])
        ys_v[sl] = plsc.load_gather(cy_v, [ids])
        return carry

    jax.lax.fori_loop(0, _BW // 16, body, 0)
    pltpu.sync_copy(xs_v, xs_hbm.at[pl.ds(base, _BW)])
    pltpu.sync_copy(ys_v, ys_hbm.at[pl.ds(base, _BW)])


def _norm_kernel(xs_ref, ys_ref, cq_ref, w_ref):
    xq = cq_ref[:, 0:1]
    yq = cq_ref[:, 1:2]
    dx = xs_ref[:, :] - xq
    dy = ys_ref[:, :] - yq
    s = dx * dx + dy * dy
    w_ref[:, :] = jnp.where(s > 0.0, jnp.sqrt(jnp.where(s > 0.0, s, 1.0)),
                            0.0)


def kernel(coords):
    n = _N
    cq = jnp.zeros((_NP, 2), jnp.float32).at[: n - 1].set(coords[1:])
    cc = jnp.zeros((_NP, 2), jnp.float32).at[:n].set(coords)
    cct = cc.T

    cx3 = cc[:, 0].reshape(_NB, 128)
    cy3 = cc[:, 1].reshape(_NB, 128)

    idx, idx0 = pl.pallas_call(
        _knn_kernel,
        grid=(_GRID,),
        in_specs=[
            pl.BlockSpec((_RT, 2), lambda i: (i, 0)),
            pl.BlockSpec((2, _NP), lambda i: (0, 0)),
            pl.BlockSpec((_NB, 128), lambda i: (0, 0)),
            pl.BlockSpec((_NB, 128), lambda i: (0, 0)),
        ],
        out_specs=[
            pl.BlockSpec((_RT, _OW), lambda i: (i, 0)),
            pl.BlockSpec((8, _OW), lambda i: (0, 0)),
        ],
        out_shape=[
            jax.ShapeDtypeStruct((_NP, _OW), jnp.int32),
            jax.ShapeDtypeStruct((8, _OW), jnp.int32),
        ],
    )(cq, cct, cx3, cy3)

    src = jnp.concatenate([idx0[0:1, :_KK], idx[: n - 1, :_KK]], axis=0)

    cx = jnp.zeros((_TBL,), jnp.float32).at[:n].set(coords[:, 0])
    cy = jnp.zeros((_TBL,), jnp.float32).at[:n].set(coords[:, 1])
    src_flat = jnp.zeros((_B,), jnp.int32).at[: n * _KK].set(
        src.reshape(-1))

    mesh = plsc.VectorSubcoreMesh(core_axis_name="c", subcore_axis_name="s")
    xs, ys = pl.kernel(
        _gather_kernel,
        out_type=[
            jax.ShapeDtypeStruct((_B,), jnp.float32),
            jax.ShapeDtypeStruct((_B,), jnp.float32),
        ],
        mesh=mesh,
        scratch_types=[
            pltpu.VMEM((_TBL,), jnp.float32),
            pltpu.VMEM((_TBL,), jnp.float32),
            pltpu.VMEM((_BW,), jnp.int32),
            pltpu.VMEM((_BW,), jnp.float32),
            pltpu.VMEM((_BW,), jnp.float32),
        ],
        compiler_params=pltpu.CompilerParams(needs_layout_passes=False),
    )(cx, cy, src_flat)

    xs2 = jnp.zeros((_NP, _KK), jnp.float32).at[:n].set(
        xs[: n * _KK].reshape(n, _KK))
    ys2 = jnp.zeros((_NP, _KK), jnp.float32).at[:n].set(
        ys[: n * _KK].reshape(n, _KK))

    weights = pl.pallas_call(
        _norm_kernel,
        grid=(_GRID,),
        in_specs=[
            pl.BlockSpec((_RT, _KK), lambda i: (i, 0)),
            pl.BlockSpec((_RT, _KK), lambda i: (i, 0)),
            pl.BlockSpec((_RT, 2), lambda i: (i, 0)),
        ],
        out_specs=pl.BlockSpec((_RT, _KK), lambda i: (i, 0)),
        out_shape=jax.ShapeDtypeStruct((_NP, _KK), jnp.float32),
    )(xs2, ys2, cc)[:n]

    dst = jnp.broadcast_to(jnp.arange(n, dtype=jnp.int32)[:, None], (n, _KK))
    edge_idx = jnp.stack([src, dst], axis=0)
    return edge_idx, weights, _KK
